# baseline (speedup 1.0000x reference)
import functools

import jax
import jax.numpy as jnp
from jax import lax
from jax.experimental import pallas as pl
from jax.experimental.pallas import tpu as pltpu
from jax.experimental.pallas import tpu_sc as plsc

NC = 2
NS = 16
NW = NC * NS
CHUNK = 128


def _make_gather(B: int, E: int, n_chunks: int, slice_row0: int):
    mesh = plsc.VectorSubcoreMesh(core_axis_name="c", subcore_axis_name="s")

    T = 2 * n_chunks
    NBUF = min(T, 6)

    @functools.partial(
        pl.kernel,
        mesh=mesh,
        out_type=(
            jax.ShapeDtypeStruct((B, E), jnp.float32),
            jax.ShapeDtypeStruct((B, E), jnp.float32),
        ),
        scratch_types=[
            pltpu.VMEM((n_chunks, CHUNK), jnp.int32),
            pltpu.VMEM((n_chunks, CHUNK), jnp.int32),
            [pltpu.VMEM((CHUNK, E), jnp.float32) for _ in range(NBUF)],
            [pltpu.SemaphoreType.DMA for _ in range(NBUF)],
            [pltpu.SemaphoreType.DMA for _ in range(NBUF)],
            pltpu.SemaphoreType.DMA,
            pltpu.SemaphoreType.DMA,
        ],
    )
    def gather(uids_hbm, iids_hbm, utab_hbm, itab_hbm, out_u, out_i,
               uidx_v, iidx_v, bufs, gsems, wsems, isem0, isem1):
        wid = lax.axis_index("s") * NC + lax.axis_index("c")
        base_row = wid * n_chunks
        cu = pltpu.async_copy(
            uids_hbm.at[pl.ds(slice_row0 + base_row, n_chunks)], uidx_v, isem0)
        ci = pltpu.async_copy(
            iids_hbm.at[pl.ds(slice_row0 + base_row, n_chunks)], iidx_v, isem1)
        cu.wait()
        ci.wait()
        steps = [(uidx_v, utab_hbm, out_u, j) for j in range(n_chunks)]
        steps += [(iidx_v, itab_hbm, out_i, j) for j in range(n_chunks)]
        copies = [
            pltpu.async_copy(tab.at[idx_v.at[j]], bufs[t % NBUF],
                             gsems[t % NBUF])
            for t, (idx_v, tab, _, j) in enumerate(steps[:NBUF])
        ]
        writes = []
        for t, (_, _, out, j) in enumerate(steps):
            copies[t].wait()
            writes.append(pltpu.async_copy(
                bufs[t % NBUF], out.at[pl.ds((base_row + j) * CHUNK, CHUNK)],
                wsems[t % NBUF]))
            nxt = t + NBUF
            if nxt < T:
                idx_v, tab, _, j2 = steps[nxt]
                writes[t].wait()
                copies.append(pltpu.async_copy(
                    tab.at[idx_v.at[j2]], bufs[nxt % NBUF], gsems[nxt % NBUF]))
        for w in writes[max(0, T - NBUF):]:
            w.wait()

    return gather


def _dot_t(x, w):
    return lax.dot_general(x, w, (((1,), (1,)), ((), ())),
                           preferred_element_type=jnp.float32)


def _mlp_body(u_ref, i_ref, w1a_ref, w1b_ref, b1_ref, w2_ref, b2_ref,
              w3_ref, b3_ref, o_ref):
    h = _dot_t(u_ref[...], w1a_ref[...])
    h = h + _dot_t(i_ref[...], w1b_ref[...])
    h = jnp.maximum(h + b1_ref[...], 0.0)
    h = jnp.maximum(_dot_t(h, w2_ref[...]) + b2_ref[...], 0.0)
    w3 = w3_ref[...]
    b3 = b3_ref[0]
    nrow = o_ref.shape[0]
    for c in range(nrow):
        o_ref[c:c + 1, :] = _dot_t(w3, h[128 * c:128 * (c + 1), :]) + b3


def _mlp_call(U, I, W1, b1, W2, b2, W3, b3, BC):
    Bs, E = U.shape
    return pl.pallas_call(
        _mlp_body,
        grid=(Bs // BC,),
        in_specs=[
            pl.BlockSpec((BC, E), lambda i: (i, 0)),
            pl.BlockSpec((BC, E), lambda i: (i, 0)),
            pl.BlockSpec((128, E), lambda i: (0, 0)),
            pl.BlockSpec((128, E), lambda i: (0, 1)),
            pl.BlockSpec((1, 128), lambda i: (0, 0)),
            pl.BlockSpec((64, 128), lambda i: (0, 0)),
            pl.BlockSpec((1, 64), lambda i: (0, 0)),
            pl.BlockSpec((1, 64), lambda i: (0, 0)),
            pl.BlockSpec(memory_space=pltpu.SMEM),
        ],
        out_specs=pl.BlockSpec((BC // 128, 128), lambda i: (i, 0)),
        out_shape=jax.ShapeDtypeStruct((Bs // 128, 128), jnp.float32),
    )(U, I, W1, W1, b1.reshape(1, 128), W2, b2.reshape(1, 64), W3, b3)


SLICES = (16384,)


def kernel(user_ids, item_ids, user_table, item_table, W1, b1, W2, b2, W3, b3):
    B = user_ids.shape[0]
    E = user_table.shape[1]

    uids = user_ids.astype(jnp.int32).reshape(B // CHUNK, CHUNK)
    iids = item_ids.astype(jnp.int32).reshape(B // CHUNK, CHUNK)
    outs = []
    row0 = 0
    for Bs in SLICES:
        n_chunks = Bs // (NW * CHUNK)
        U, I = _make_gather(Bs, E, n_chunks, row0)(
            uids, iids, user_table, item_table)
        outs.append(_mlp_call(U, I, W1, b1, W2, b2, W3, b3, min(2048, Bs // 2)))
        row0 += Bs // CHUNK
    return jnp.concatenate(outs, axis=0).reshape(B)


# speedup vs baseline: 1.0234x; 1.0234x over previous
import functools

import jax
import jax.numpy as jnp
from jax import lax
from jax.experimental import pallas as pl
from jax.experimental.pallas import tpu as pltpu
from jax.experimental.pallas import tpu_sc as plsc

NC = 2
NS = 16
NW = NC * NS
CHUNK = 128


def _make_gather(B: int, E: int, n_chunks: int, slice_row0: int):
    mesh = plsc.VectorSubcoreMesh(core_axis_name="c", subcore_axis_name="s")

    T = 2 * n_chunks
    NBUF = min(T, 6)

    @functools.partial(
        pl.kernel,
        mesh=mesh,
        out_type=(
            jax.ShapeDtypeStruct((B, E), jnp.float32),
            jax.ShapeDtypeStruct((B, E), jnp.float32),
        ),
        scratch_types=[
            pltpu.VMEM((n_chunks, CHUNK), jnp.int32),
            pltpu.VMEM((n_chunks, CHUNK), jnp.int32),
            [pltpu.VMEM((CHUNK, E), jnp.float32) for _ in range(NBUF)],
            [pltpu.SemaphoreType.DMA for _ in range(NBUF)],
            [pltpu.SemaphoreType.DMA for _ in range(NBUF)],
            pltpu.SemaphoreType.DMA,
            pltpu.SemaphoreType.DMA,
        ],
    )
    def gather(uids_hbm, iids_hbm, utab_hbm, itab_hbm, out_u, out_i,
               uidx_v, iidx_v, bufs, gsems, wsems, isem0, isem1):
        wid = lax.axis_index("s") * NC + lax.axis_index("c")
        base_row = wid * n_chunks
        cu = pltpu.async_copy(
            uids_hbm.at[pl.ds(slice_row0 + base_row, n_chunks)], uidx_v, isem0)
        ci = pltpu.async_copy(
            iids_hbm.at[pl.ds(slice_row0 + base_row, n_chunks)], iidx_v, isem1)
        cu.wait()
        ci.wait()
        steps = [(uidx_v, utab_hbm, out_u, j) for j in range(n_chunks)]
        steps += [(iidx_v, itab_hbm, out_i, j) for j in range(n_chunks)]
        copies = [
            pltpu.async_copy(tab.at[idx_v.at[j]], bufs[t % NBUF],
                             gsems[t % NBUF])
            for t, (idx_v, tab, _, j) in enumerate(steps[:NBUF])
        ]
        writes = []
        for t, (_, _, out, j) in enumerate(steps):
            copies[t].wait()
            writes.append(pltpu.async_copy(
                bufs[t % NBUF], out.at[pl.ds((base_row + j) * CHUNK, CHUNK)],
                wsems[t % NBUF]))
            nxt = t + NBUF
            if nxt < T:
                idx_v, tab, _, j2 = steps[nxt]
                writes[t].wait()
                copies.append(pltpu.async_copy(
                    tab.at[idx_v.at[j2]], bufs[nxt % NBUF], gsems[nxt % NBUF]))
        for w in writes[max(0, T - NBUF):]:
            w.wait()

    return gather


def _dot_t(x, w):
    return lax.dot_general(x, w, (((1,), (1,)), ((), ())),
                           preferred_element_type=jnp.float32)


def _mlp_body(u_ref, i_ref, w1a_ref, w1b_ref, b1_ref, w2_ref, b2_ref,
              w3_ref, b3_ref, o_ref):
    h = _dot_t(u_ref[...], w1a_ref[...])
    h = h + _dot_t(i_ref[...], w1b_ref[...])
    h = jnp.maximum(h + b1_ref[...], 0.0)
    h = jnp.maximum(_dot_t(h, w2_ref[...]) + b2_ref[...], 0.0)
    w3 = w3_ref[...]
    b3 = b3_ref[0]
    nrow = o_ref.shape[0]
    for c in range(nrow):
        o_ref[c:c + 1, :] = _dot_t(w3, h[128 * c:128 * (c + 1), :]) + b3


def _mlp_call(U, I, W1, b1, W2, b2, W3, b3, BC):
    Bs, E = U.shape
    return pl.pallas_call(
        _mlp_body,
        grid=(Bs // BC,),
        in_specs=[
            pl.BlockSpec((BC, E), lambda i: (i, 0)),
            pl.BlockSpec((BC, E), lambda i: (i, 0)),
            pl.BlockSpec((128, E), lambda i: (0, 0)),
            pl.BlockSpec((128, E), lambda i: (0, 1)),
            pl.BlockSpec((1, 128), lambda i: (0, 0)),
            pl.BlockSpec((64, 128), lambda i: (0, 0)),
            pl.BlockSpec((1, 64), lambda i: (0, 0)),
            pl.BlockSpec((1, 64), lambda i: (0, 0)),
            pl.BlockSpec(memory_space=pltpu.SMEM),
        ],
        out_specs=pl.BlockSpec((BC // 128, 128), lambda i: (i, 0)),
        out_shape=jax.ShapeDtypeStruct((Bs // 128, 128), jnp.float32),
    )(U, I, W1, W1, b1.reshape(1, 128), W2, b2.reshape(1, 64), W3, b3)


SLICES = (16384,)


def kernel(user_ids, item_ids, user_table, item_table, W1, b1, W2, b2, W3, b3):
    B = user_ids.shape[0]
    E = user_table.shape[1]

    uids = user_ids.astype(jnp.int32).reshape(B // CHUNK, CHUNK)
    iids = item_ids.astype(jnp.int32).reshape(B // CHUNK, CHUNK)
    outs = []
    row0 = 0
    for Bs in SLICES:
        n_chunks = Bs // (NW * CHUNK)
        U, I = _make_gather(Bs, E, n_chunks, row0)(
            uids, iids, user_table, item_table)
        outs.append(_mlp_call(U, I, W1, b1, W2, b2, W3, b3, min(8192, Bs // 2)))
        row0 += Bs // CHUNK
    return jnp.concatenate(outs, axis=0).reshape(B)


# speedup vs baseline: 1.0371x; 1.0134x over previous
import functools

import jax
import jax.numpy as jnp
from jax import lax
from jax.experimental import pallas as pl
from jax.experimental.pallas import tpu as pltpu
from jax.experimental.pallas import tpu_sc as plsc

NC = 2
NS = 16
NW = NC * NS
CHUNK = 128


def _make_gather(B: int, E: int, n_chunks: int, slice_row0: int):
    mesh = plsc.VectorSubcoreMesh(core_axis_name="c", subcore_axis_name="s")

    T = 2 * n_chunks
    NBUF = min(T, 6)

    @functools.partial(
        pl.kernel,
        mesh=mesh,
        out_type=(
            jax.ShapeDtypeStruct((B, E), jnp.float32),
            jax.ShapeDtypeStruct((B, E), jnp.float32),
        ),
        scratch_types=[
            pltpu.VMEM((n_chunks, CHUNK), jnp.int32),
            pltpu.VMEM((n_chunks, CHUNK), jnp.int32),
            [pltpu.VMEM((CHUNK, E), jnp.float32) for _ in range(NBUF)],
            [pltpu.SemaphoreType.DMA for _ in range(NBUF)],
            [pltpu.SemaphoreType.DMA for _ in range(NBUF)],
            pltpu.SemaphoreType.DMA,
            pltpu.SemaphoreType.DMA,
        ],
    )
    def gather(uids_hbm, iids_hbm, utab_hbm, itab_hbm, out_u, out_i,
               uidx_v, iidx_v, bufs, gsems, wsems, isem0, isem1):
        wid = lax.axis_index("s") * NC + lax.axis_index("c")
        base_row = wid * n_chunks
        cu = pltpu.async_copy(
            uids_hbm.at[pl.ds(slice_row0 + base_row, n_chunks)], uidx_v, isem0)
        ci = pltpu.async_copy(
            iids_hbm.at[pl.ds(slice_row0 + base_row, n_chunks)], iidx_v, isem1)
        cu.wait()
        ci.wait()
        steps = [(uidx_v, utab_hbm, out_u, j) for j in range(n_chunks)]
        steps += [(iidx_v, itab_hbm, out_i, j) for j in range(n_chunks)]
        copies = [
            pltpu.async_copy(tab.at[idx_v.at[j]], bufs[t % NBUF],
                             gsems[t % NBUF])
            for t, (idx_v, tab, _, j) in enumerate(steps[:NBUF])
        ]
        writes = []
        for t, (_, _, out, j) in enumerate(steps):
            copies[t].wait()
            writes.append(pltpu.async_copy(
                bufs[t % NBUF], out.at[pl.ds((base_row + j) * CHUNK, CHUNK)],
                wsems[t % NBUF]))
            nxt = t + NBUF
            if nxt < T:
                idx_v, tab, _, j2 = steps[nxt]
                writes[t].wait()
                copies.append(pltpu.async_copy(
                    tab.at[idx_v.at[j2]], bufs[nxt % NBUF], gsems[nxt % NBUF]))
        for w in writes[max(0, T - NBUF):]:
            w.wait()

    return gather


def _dot_t(x, w):
    return lax.dot_general(x, w, (((1,), (1,)), ((), ())),
                           preferred_element_type=jnp.float32)


def _mlp_body(u_ref, i_ref, w1a_ref, w1b_ref, b1_ref, w2_ref, b2_ref,
              w3_ref, b3_ref, o_ref):
    h = _dot_t(u_ref[...], w1a_ref[...])
    h = h + _dot_t(i_ref[...], w1b_ref[...])
    h = jnp.maximum(h + b1_ref[...], 0.0)
    h = jnp.maximum(_dot_t(h, w2_ref[...]) + b2_ref[...], 0.0)
    w3 = w3_ref[...]
    b3 = b3_ref[0]
    nrow = o_ref.shape[0]
    for c in range(nrow):
        o_ref[c:c + 1, :] = _dot_t(w3, h[128 * c:128 * (c + 1), :]) + b3


def _mlp_call(U, I, W1, b1, W2, b2, W3, b3, BC):
    Bs, E = U.shape
    return pl.pallas_call(
        _mlp_body,
        grid=(Bs // BC,),
        in_specs=[
            pl.BlockSpec((BC, E), lambda i: (i, 0)),
            pl.BlockSpec((BC, E), lambda i: (i, 0)),
            pl.BlockSpec((128, E), lambda i: (0, 0)),
            pl.BlockSpec((128, E), lambda i: (0, 1)),
            pl.BlockSpec((1, 128), lambda i: (0, 0)),
            pl.BlockSpec((64, 128), lambda i: (0, 0)),
            pl.BlockSpec((1, 64), lambda i: (0, 0)),
            pl.BlockSpec((1, 64), lambda i: (0, 0)),
            pl.BlockSpec(memory_space=pltpu.SMEM),
        ],
        out_specs=pl.BlockSpec((BC // 128, 128), lambda i: (i, 0)),
        out_shape=jax.ShapeDtypeStruct((Bs // 128, 128), jnp.float32),
    )(U, I, W1, W1, b1.reshape(1, 128), W2, b2.reshape(1, 64), W3, b3)


SLICES = (16384,)


def kernel(user_ids, item_ids, user_table, item_table, W1, b1, W2, b2, W3, b3):
    B = user_ids.shape[0]
    E = user_table.shape[1]

    uids = user_ids.astype(jnp.int32).reshape(B // CHUNK, CHUNK)
    iids = item_ids.astype(jnp.int32).reshape(B // CHUNK, CHUNK)
    outs = []
    row0 = 0
    for Bs in SLICES:
        n_chunks = Bs // (NW * CHUNK)
        U, I = _make_gather(Bs, E, n_chunks, row0)(
            uids, iids, user_table, item_table)
        outs.append(_mlp_call(U, I, W1, b1, W2, b2, W3, b3, min(4096, Bs // 2)))
        row0 += Bs // CHUNK
    return jnp.concatenate(outs, axis=0).reshape(B)
